# Initial kernel scaffold; baseline (speedup 1.0000x reference)
import jax
import jax.numpy as jnp
from jax.experimental import pallas as pl


def kernel(x, edge_index, W1, b1, W2, b2, Wf, bf):
    raise NotImplementedError("write your pallas kernel here")


# speedup vs baseline: 9.4155x; 9.4155x over previous
import functools

import jax
import jax.numpy as jnp
from jax import lax
from jax.experimental import pallas as pl
from jax.experimental.pallas import tpu as pltpu
from jax.experimental.pallas import tpu_sc as plsc

N_NODES = 10000
N_EDGES = 320000
D_IN = 128
D_HID = 64

NC = 2
NS = 16
CHUNK_B = 128
CHUNKS_B = 80
EDGES_PER_TILE_B = CHUNK_B * CHUNKS_B
E_PAD = NC * NS * EDGES_PER_TILE_B
DUMP_ROW = N_NODES
ROWS_PER_TILE = 632
ACC_ROWS = NS * ROWS_PER_TILE

EDGES_PER_TILE_D = N_EDGES // NS
CHUNKS_D = EDGES_PER_TILE_D // 16
V_LEN = 10240
WORDS_PER_TILE_D = V_LEN // NS


def _stage_a(x, W1, b1):
    def body(x_ref, w_ref, b_ref, o_ref):
        acc = jnp.dot(x_ref[...], w_ref[...], preferred_element_type=jnp.float32)
        o_ref[...] = jnp.maximum(acc + b_ref[...], 0.0)

    return pl.pallas_call(
        body,
        out_shape=jax.ShapeDtypeStruct((N_NODES, D_HID), jnp.float32),
    )(x, W1, b1.reshape(1, D_HID))


def _stage_c(p, W2, b2, Wf):
    def body(p_ref, w2_ref, b2_ref, wf_ref, o_ref):
        h1 = p_ref[0] + p_ref[1]
        m2 = jnp.maximum(
            jnp.dot(h1, w2_ref[...], preferred_element_type=jnp.float32)
            + b2_ref[...], 0.0)
        o_ref[...] = jnp.dot(m2, wf_ref[...], preferred_element_type=jnp.float32)

    return pl.pallas_call(
        body,
        out_shape=jax.ShapeDtypeStruct((N_NODES, 1), jnp.float32),
    )(p, W2, b2.reshape(1, D_HID), Wf)


def _stage_b(m1, src3, dst3, zrows):
    mesh = plsc.VectorSubcoreMesh(core_axis_name="c", subcore_axis_name="s")

    @functools.partial(
        pl.kernel,
        mesh=mesh,
        compiler_params=pltpu.CompilerParams(use_tc_tiling_on_sc=False),
        out_type=jax.ShapeDtypeStruct((NC, ACC_ROWS, D_HID), jnp.float32),
        scratch_types=[
            pltpu.VMEM((CHUNKS_B, CHUNK_B), jnp.int32),
            pltpu.VMEM((CHUNKS_B, CHUNK_B), jnp.int32),
            pltpu.VMEM((CHUNK_B, D_HID), jnp.float32),
            pltpu.VMEM_SHARED((ACC_ROWS, D_HID), jnp.float32),
            pltpu.SemaphoreType.DMA,
        ],
    )
    def k(m1_hbm, src_hbm, dst_hbm, z_hbm, out_hbm, src_t, dst_t, buf, acc, gsem):
        cid = lax.axis_index("c")
        sid = lax.axis_index("s")
        wid = cid * NS + sid
        pltpu.sync_copy(src_hbm.at[wid], src_t)
        pltpu.sync_copy(dst_hbm.at[wid], dst_t)
        rbase = sid * ROWS_PER_TILE
        pltpu.sync_copy(z_hbm, acc.at[pl.ds(rbase, ROWS_PER_TILE)])
        plsc.subcore_barrier()

        def body(c, carry):
            pltpu.async_copy(m1_hbm.at[src_t.at[c]], buf, gsem).wait()
            pltpu.sync_copy(buf, acc.at[dst_t.at[c]], add=True)
            return carry

        lax.fori_loop(0, CHUNKS_B, body, 0)
        plsc.subcore_barrier()
        pltpu.sync_copy(acc.at[pl.ds(rbase, ROWS_PER_TILE)],
                        out_hbm.at[cid, pl.ds(rbase, ROWS_PER_TILE)])

    return k(m1, src3, dst3, zrows)


def _stage_d(v, src2, dst2, bfv):
    mesh = plsc.VectorSubcoreMesh(core_axis_name="c", subcore_axis_name="s")

    @functools.partial(
        pl.kernel,
        mesh=mesh,
        compiler_params=pltpu.CompilerParams(needs_layout_passes=False),
        out_type=jax.ShapeDtypeStruct((V_LEN,), jnp.float32),
        scratch_types=[
            pltpu.VMEM((EDGES_PER_TILE_D,), jnp.int32),
            pltpu.VMEM((EDGES_PER_TILE_D,), jnp.int32),
            pltpu.VMEM((V_LEN,), jnp.float32),
            pltpu.VMEM((V_LEN,), jnp.float32),
            pltpu.VMEM((NS, WORDS_PER_TILE_D), jnp.float32),
            pltpu.VMEM((16,), jnp.float32),
            pltpu.VMEM_SHARED((NS, V_LEN), jnp.float32),
        ],
    )
    def k(v_hbm, src_hbm, dst_hbm, bf_hbm, out_hbm,
          src_t, dst_t, vloc, acc, cbuf, bfb, slots):
        cid = lax.axis_index("c")
        sid = lax.axis_index("s")

        @pl.when(cid == 0)
        def _():
            pltpu.sync_copy(src_hbm.at[sid], src_t)
            pltpu.sync_copy(dst_hbm.at[sid], dst_t)
            pltpu.sync_copy(v_hbm, vloc)
            pltpu.sync_copy(bf_hbm, bfb)

            def zbody(j, carry):
                acc[pl.ds(j * 16, 16)] = jnp.zeros((16,), jnp.float32)
                return carry

            lax.fori_loop(0, V_LEN // 16, zbody, 0)

            def body(i, carry):
                s16 = src_t[pl.ds(i * 16, 16)]
                d16 = dst_t[pl.ds(i * 16, 16)]
                vals = plsc.load_gather(vloc, [s16])
                plsc.addupdate_scatter(acc, [d16], vals)
                return carry

            lax.fori_loop(0, CHUNKS_D, body, 0)
            pltpu.sync_copy(acc, slots.at[sid])
            plsc.subcore_barrier()
            wbase = sid * WORDS_PER_TILE_D
            pltpu.sync_copy(slots.at[:, pl.ds(wbase, WORDS_PER_TILE_D)], cbuf)

            def rbody(j, carry):
                tot = bfb[...]
                for s in range(NS):
                    tot = tot + cbuf[s, pl.ds(j * 16, 16)]
                cbuf[0, pl.ds(j * 16, 16)] = tot
                return carry

            lax.fori_loop(0, WORDS_PER_TILE_D // 16, rbody, 0)
            pltpu.sync_copy(cbuf.at[0], out_hbm.at[pl.ds(wbase, WORDS_PER_TILE_D)])

    return k(v, src2, dst2, bfv)


def kernel(x, edge_index, W1, b1, W2, b2, Wf, bf):
    src = edge_index[0].astype(jnp.int32)
    dst = edge_index[1].astype(jnp.int32)

    pad = E_PAD - N_EDGES
    src_b = jnp.concatenate([src, jnp.zeros((pad,), jnp.int32)])
    dst_b = jnp.concatenate([dst, jnp.full((pad,), DUMP_ROW, jnp.int32)])
    src3 = src_b.reshape(NC * NS, CHUNKS_B, CHUNK_B)
    dst3 = dst_b.reshape(NC * NS, CHUNKS_B, CHUNK_B)
    zrows = jnp.zeros((ROWS_PER_TILE, D_HID), jnp.float32)

    m1 = _stage_a(x, W1, b1)
    p = _stage_b(m1, src3, dst3, zrows)
    v = _stage_c(p[:, :N_NODES], W2, b2, Wf)

    v_pad = jnp.concatenate([v[:, 0], jnp.zeros((V_LEN - N_NODES,), jnp.float32)])
    src2 = src.reshape(NS, EDGES_PER_TILE_D)
    dst2 = dst.reshape(NS, EDGES_PER_TILE_D)
    bfv = jnp.broadcast_to(bf, (16,))
    out = _stage_d(v_pad, src2, dst2, bfv)
    return out[:N_NODES].reshape(N_NODES, 1)
